# Initial kernel scaffold; baseline (speedup 1.0000x reference)
import jax
import jax.numpy as jnp
from jax.experimental import pallas as pl


def kernel(y, label, power_ratio, power_consumption):
    raise NotImplementedError("write your pallas kernel here")


# speedup vs baseline: 3.3739x; 3.3739x over previous
import jax
import jax.numpy as jnp
from jax.experimental import pallas as pl
from jax.experimental.pallas import tpu as pltpu

_F, _N, _B, _C = 2, 2, 16384, 1000
_M = 0.3
_T = 0.1
_LAMBDA_P = 0.1
_RHO = 0.01

_BM = 512
_NB = _B // _BM
_NS = _F * _N


def _loss_body(y_ref, lab_ref, pc_ref, out_ref):
    s = pl.program_id(0)
    j = pl.program_id(1)

    @pl.when((s == 0) & (j == 0))
    def _init():
        out_ref[0, 0] = 0.0

    yb = y_ref[0]
    lab = lab_ref[0]
    cols = jax.lax.broadcasted_iota(jnp.int32, (_BM, _C), 1)
    mask = cols == lab
    fy = jnp.sum(jnp.where(mask, yb, 0.0), axis=1, keepdims=True)
    fnym = jnp.max(jnp.where(mask, -1e10, yb), axis=1, keepdims=True)
    l = jnp.maximum(_M + _T - fy, 0.0) + jnp.maximum(_M + fnym, 0.0)
    out_ref[0, 0] += jnp.sum(l) * (1.0 / (_NS * _B))

    @pl.when((s == _NS - 1) & (j == _NB - 1))
    def _fini():
        pc = pc_ref[0, 0]
        out_ref[0, 0] += _LAMBDA_P * pc + (_RHO / 2.0) * pc * pc


def kernel(y, label, power_ratio, power_consumption):
    del power_ratio
    y4 = y.reshape(_NS, _B, _C)
    lab3 = label.reshape(_NB, _BM, 1)
    pc = power_consumption.reshape(1, 1)

    out = pl.pallas_call(
        _loss_body,
        grid=(_NS, _NB),
        in_specs=[
            pl.BlockSpec((1, _BM, _C), lambda s, j: (s, j, 0)),
            pl.BlockSpec((1, _BM, 1), lambda s, j: (j, 0, 0)),
            pl.BlockSpec(memory_space=pltpu.SMEM),
        ],
        out_specs=pl.BlockSpec(memory_space=pltpu.SMEM),
        out_shape=jax.ShapeDtypeStruct((1, 1), jnp.float32),
        compiler_params=pltpu.CompilerParams(
            dimension_semantics=("arbitrary", "arbitrary"),
        ),
    )(y4, lab3, pc)
    return out.reshape(1)
